# baseline (speedup 1.0000x reference)
import functools

import jax
import jax.numpy as jnp
from jax import lax
from jax.experimental import pallas as pl
from jax.experimental.pallas import tpu as pltpu
from jax.experimental.pallas import tpu_sc as plsc

_N = 10000
_D = 128
_E = 320000
_NC = 2
_NS = 16
_NW = _NC * _NS
_L = 16
_CHUNK = 125
_NCHUNKS = 80
_EDGES_PER_TILE = _E // _NW
_RCHUNK = 96
_RFULL = _N // _RCHUNK
_RTAIL = _N - _RFULL * _RCHUNK


def _unpack_chunk(pack_v, j, sidx, didx):
    offs = [l * _L for l in range(_CHUNK // _L)] + [_CHUNK - _L]
    for o in offs:
        w = pack_v[pl.ds(j * _CHUNK + o, _L)]
        sidx[0, pl.ds(o, _L)] = lax.shift_right_logical(w, 16)
        didx[0, pl.ds(o, _L)] = lax.bitwise_and(w, 0xFFFF)


def _sc_body(x_hbm, pack_hbm, zeros_hbm, out_hbm,
             pack_v, sidx, didx, rows, agg_sh, sem_g, sem_s):
    c = lax.axis_index("c")
    s = lax.axis_index("s")
    wid = c * _NS + s
    n = _NCHUNKS

    pbase = pl.multiple_of(wid * _EDGES_PER_TILE, 16)
    pltpu.sync_copy(pack_hbm.at[pl.ds(pbase, _EDGES_PER_TILE)], pack_v)

    pltpu.sync_copy(zeros_hbm, rows[0])
    for k in range(-(-(_RFULL + 1) // _NS)):
        blk = s + k * _NS

        @pl.when(blk < _RFULL)
        def _():
            pltpu.sync_copy(rows[0].at[pl.ds(0, _RCHUNK)],
                            agg_sh.at[pl.ds(blk * _RCHUNK, _RCHUNK)])

        @pl.when(blk == _RFULL)
        def _():
            pltpu.sync_copy(rows[0].at[pl.ds(0, _RTAIL)],
                            agg_sh.at[pl.ds(blk * _RCHUNK, _RTAIL)])

    plsc.subcore_barrier()

    def start_gather(j, b):
        _unpack_chunk(pack_v, j, sidx[b], didx[b])
        pltpu.async_copy(x_hbm.at[sidx[b].at[0]], rows[b], sem_g[b])

    def wait_gather(b):
        pltpu.make_async_copy(x_hbm.at[sidx[b].at[0]], rows[b], sem_g[b]).wait()

    start_gather(0, 0)
    start_gather(1, 1)

    def pair_body(i, carry):
        j0 = 2 * i
        for bi in range(2):
            j = j0 + bi
            wait_gather(bi)
            pltpu.async_copy(rows[bi], agg_sh.at[didx[bi].at[0]],
                             sem_s, add=True).wait()

            @pl.when(j + 2 < n)
            def _():
                start_gather(j + 2, bi)

        return carry

    lax.fori_loop(0, n // 2, pair_body, 0, unroll=False)

    plsc.subcore_barrier()

    for k in range(-(-(_RFULL + 1) // _NS)):
        blk = s + k * _NS
        r0 = pl.multiple_of(blk * _RCHUNK, 16)
        o0 = pl.multiple_of(c * _N + r0, 16)

        @pl.when(blk < _RFULL)
        def _():
            pltpu.sync_copy(agg_sh.at[pl.ds(r0, _RCHUNK)],
                            rows[0].at[pl.ds(0, _RCHUNK)])
            pltpu.sync_copy(rows[0].at[pl.ds(0, _RCHUNK)],
                            out_hbm.at[pl.ds(o0, _RCHUNK)])

        @pl.when(blk == _RFULL)
        def _():
            pltpu.sync_copy(agg_sh.at[pl.ds(r0, _RTAIL)],
                            rows[0].at[pl.ds(0, _RTAIL)])
            pltpu.sync_copy(rows[0].at[pl.ds(0, _RTAIL)],
                            out_hbm.at[pl.ds(o0, _RTAIL)])


@functools.cache
def _sc_segment_sum():
    mesh = plsc.VectorSubcoreMesh(
        core_axis_name="c", subcore_axis_name="s", num_cores=_NC, num_subcores=_NS
    )
    return pl.kernel(
        _sc_body,
        out_type=jax.ShapeDtypeStruct((_NC * _N, _D), jnp.bfloat16),
        mesh=mesh,
        compiler_params=pltpu.CompilerParams(use_tc_tiling_on_sc=False),
        scratch_types=[
            pltpu.VMEM((_EDGES_PER_TILE,), jnp.int32),
            [pltpu.VMEM((1, _CHUNK), jnp.int32) for _ in range(2)],
            [pltpu.VMEM((1, _CHUNK), jnp.int32) for _ in range(2)],
            [pltpu.VMEM((_CHUNK, _D), jnp.bfloat16) for _ in range(2)],
            pltpu.VMEM_SHARED((_N, _D), jnp.bfloat16),
            [pltpu.SemaphoreType.DMA for _ in range(2)],
            pltpu.SemaphoreType.DMA,
        ],
    )


_TC_ROWS = 1000


def _pack_body(e_ref, o_ref):
    o_ref[...] = jnp.left_shift(e_ref[0], 16) | e_ref[1]


def _tc_pack(edge_index):
    return pl.pallas_call(
        _pack_body,
        grid=(1,),
        in_specs=[pl.BlockSpec((2, _E), lambda i: (0, 0))],
        out_specs=pl.BlockSpec((_E,), lambda i: (0,)),
        out_shape=jax.ShapeDtypeStruct((_E,), jnp.int32),
    )(edge_index)


def _self_body(x_ref, ws_ref, b_ref, o_ref):
    o_ref[...] = jnp.dot(x_ref[...], ws_ref[...],
                         preferred_element_type=jnp.float32) + b_ref[...]


def _tc_self(x, W_self, b2):
    return pl.pallas_call(
        _self_body,
        grid=(_N // _TC_ROWS,),
        in_specs=[
            pl.BlockSpec((_TC_ROWS, _D), lambda i: (i, 0)),
            pl.BlockSpec((_D, _D), lambda i: (0, 0)),
            pl.BlockSpec((1, _D), lambda i: (0, 0)),
        ],
        out_specs=pl.BlockSpec((_TC_ROWS, _D), lambda i: (i, 0)),
        out_shape=jax.ShapeDtypeStruct((_N, _D), jnp.float32),
    )(x, W_self, b2)


def _final_body(sp_ref, agg0_ref, agg1_ref, wm_ref, o_ref):
    agg = agg0_ref[...].astype(jnp.float32) + agg1_ref[...].astype(jnp.float32)
    acc = sp_ref[...] + jnp.dot(agg, wm_ref[...],
                                preferred_element_type=jnp.float32)
    o_ref[...] = jnp.maximum(acc, 0.0)


_NBLK = _N // _TC_ROWS


def _tc_final(selfpart, agg_flat, W_msg):
    return pl.pallas_call(
        _final_body,
        grid=(_NBLK,),
        in_specs=[
            pl.BlockSpec((_TC_ROWS, _D), lambda i: (i, 0)),
            pl.BlockSpec((_TC_ROWS, _D), lambda i: (i, 0)),
            pl.BlockSpec((_TC_ROWS, _D), lambda i: (i + _NBLK, 0)),
            pl.BlockSpec((_D, _D), lambda i: (0, 0)),
        ],
        out_specs=pl.BlockSpec((_TC_ROWS, _D), lambda i: (i, 0)),
        out_shape=jax.ShapeDtypeStruct((_N, _D), jnp.float32),
    )(selfpart, agg_flat, agg_flat, W_msg)


def _cast_body(x_ref, o_ref):
    o_ref[...] = x_ref[...].astype(jnp.bfloat16)


def _tc_cast(x):
    return pl.pallas_call(
        _cast_body,
        grid=(_NBLK,),
        in_specs=[pl.BlockSpec((_TC_ROWS, _D), lambda i: (i, 0))],
        out_specs=pl.BlockSpec((_TC_ROWS, _D), lambda i: (i, 0)),
        out_shape=jax.ShapeDtypeStruct((_N, _D), jnp.bfloat16),
    )(x)


def kernel(x, edge_index, W_msg, W_self, b):
    packed = _tc_pack(edge_index.astype(jnp.int32))
    zeros_blk = jnp.zeros((_CHUNK, _D), jnp.bfloat16)
    xbf = _tc_cast(x)
    agg_flat = _sc_segment_sum()(xbf, packed, zeros_blk)
    selfpart = _tc_self(x, W_self, b.reshape(1, _D))
    return _tc_final(selfpart, agg_flat, W_msg)


# speedup vs baseline: 1.0636x; 1.0636x over previous
import functools

import jax
import jax.numpy as jnp
from jax import lax
from jax.experimental import pallas as pl
from jax.experimental.pallas import tpu as pltpu
from jax.experimental.pallas import tpu_sc as plsc

_N = 10000
_D = 128
_E = 320000
_NC = 2
_NS = 16
_NW = _NC * _NS
_L = 16
_CHUNK = 125
_NCHUNKS = 80
_EDGES_PER_TILE = _E // _NW
_RCHUNK = 96
_RFULL = _N // _RCHUNK
_RTAIL = _N - _RFULL * _RCHUNK


def _unpack_chunk(pack_v, j, sidx, didx):
    offs = [l * _L for l in range(_CHUNK // _L)] + [_CHUNK - _L]
    for o in offs:
        w = pack_v[pl.ds(j * _CHUNK + o, _L)]
        sidx[0, pl.ds(o, _L)] = lax.shift_right_logical(w, 16)
        didx[0, pl.ds(o, _L)] = lax.bitwise_and(w, 0xFFFF)


def _sc_body(x_hbm, pack_hbm, zeros_hbm, out_hbm,
             pack_v, sidx, didx, rows, agg_sh, sem_g, sem_s):
    c = lax.axis_index("c")
    s = lax.axis_index("s")
    wid = c * _NS + s
    n = _NCHUNKS

    pbase = pl.multiple_of(wid * _EDGES_PER_TILE, 16)
    pltpu.sync_copy(pack_hbm.at[pl.ds(pbase, _EDGES_PER_TILE)], pack_v)

    pltpu.sync_copy(zeros_hbm, rows[0])
    for k in range(-(-(_RFULL + 1) // _NS)):
        blk = s + k * _NS

        @pl.when(blk < _RFULL)
        def _():
            pltpu.sync_copy(rows[0].at[pl.ds(0, _RCHUNK)],
                            agg_sh.at[pl.ds(blk * _RCHUNK, _RCHUNK)])

        @pl.when(blk == _RFULL)
        def _():
            pltpu.sync_copy(rows[0].at[pl.ds(0, _RTAIL)],
                            agg_sh.at[pl.ds(blk * _RCHUNK, _RTAIL)])

    plsc.subcore_barrier()

    def start_gather(j, b):
        _unpack_chunk(pack_v, j, sidx[b], didx[b])
        pltpu.async_copy(x_hbm.at[sidx[b].at[0]], rows[b], sem_g[b])

    def wait_gather(b):
        pltpu.make_async_copy(x_hbm.at[sidx[b].at[0]], rows[b], sem_g[b]).wait()

    start_gather(0, 0)
    start_gather(1, 1)

    def pair_body(i, carry):
        j0 = 2 * i
        for bi in range(2):
            j = j0 + bi
            wait_gather(bi)
            pltpu.async_copy(rows[bi], agg_sh.at[didx[bi].at[0]],
                             sem_s, add=True).wait()

            @pl.when(j + 2 < n)
            def _():
                start_gather(j + 2, bi)

        return carry

    lax.fori_loop(0, n // 2, pair_body, 0, unroll=False)

    plsc.subcore_barrier()

    for k in range(-(-(_RFULL + 1) // _NS)):
        blk = s + k * _NS
        r0 = pl.multiple_of(blk * _RCHUNK, 16)
        o0 = pl.multiple_of(c * _N + r0, 16)

        @pl.when(blk < _RFULL)
        def _():
            pltpu.sync_copy(agg_sh.at[pl.ds(r0, _RCHUNK)],
                            rows[0].at[pl.ds(0, _RCHUNK)])
            pltpu.sync_copy(rows[0].at[pl.ds(0, _RCHUNK)],
                            out_hbm.at[pl.ds(o0, _RCHUNK)])

        @pl.when(blk == _RFULL)
        def _():
            pltpu.sync_copy(agg_sh.at[pl.ds(r0, _RTAIL)],
                            rows[0].at[pl.ds(0, _RTAIL)])
            pltpu.sync_copy(rows[0].at[pl.ds(0, _RTAIL)],
                            out_hbm.at[pl.ds(o0, _RTAIL)])


@functools.cache
def _sc_segment_sum():
    mesh = plsc.VectorSubcoreMesh(
        core_axis_name="c", subcore_axis_name="s", num_cores=_NC, num_subcores=_NS
    )
    return pl.kernel(
        _sc_body,
        out_type=jax.ShapeDtypeStruct((_NC * _N, _D), jnp.bfloat16),
        mesh=mesh,
        compiler_params=pltpu.CompilerParams(use_tc_tiling_on_sc=False),
        scratch_types=[
            pltpu.VMEM((_EDGES_PER_TILE,), jnp.int32),
            [pltpu.VMEM((1, _CHUNK), jnp.int32) for _ in range(2)],
            [pltpu.VMEM((1, _CHUNK), jnp.int32) for _ in range(2)],
            [pltpu.VMEM((_CHUNK, _D), jnp.bfloat16) for _ in range(2)],
            pltpu.VMEM_SHARED((_N, _D), jnp.bfloat16),
            [pltpu.SemaphoreType.DMA for _ in range(2)],
            pltpu.SemaphoreType.DMA,
        ],
    )


_TC_ROWS = 1000


def _pack_body(e_ref, o_ref):
    o_ref[...] = jnp.left_shift(e_ref[0], 16) | e_ref[1]


def _tc_pack(edge_index):
    return pl.pallas_call(
        _pack_body,
        grid=(1,),
        in_specs=[pl.BlockSpec((2, _E), lambda i: (0, 0))],
        out_specs=pl.BlockSpec((_E,), lambda i: (0,)),
        out_shape=jax.ShapeDtypeStruct((_E,), jnp.int32),
    )(edge_index)


def _self_body(x_ref, ws_ref, b_ref, o_ref):
    o_ref[...] = jnp.dot(x_ref[...], ws_ref[...],
                         preferred_element_type=jnp.float32) + b_ref[...]


def _tc_self(x, W_self, b2):
    return pl.pallas_call(
        _self_body,
        grid=(_N // _TC_ROWS,),
        in_specs=[
            pl.BlockSpec((_TC_ROWS, _D), lambda i: (i, 0)),
            pl.BlockSpec((_D, _D), lambda i: (0, 0)),
            pl.BlockSpec((1, _D), lambda i: (0, 0)),
        ],
        out_specs=pl.BlockSpec((_TC_ROWS, _D), lambda i: (i, 0)),
        out_shape=jax.ShapeDtypeStruct((_N, _D), jnp.float32),
    )(x, W_self, b2)


def _final_body(sp_ref, agg_ref, wm_ref, o_ref):
    agg = agg_ref[0].astype(jnp.float32) + agg_ref[1].astype(jnp.float32)
    acc = sp_ref[...] + jnp.dot(agg, wm_ref[...],
                                preferred_element_type=jnp.float32)
    o_ref[...] = jnp.maximum(acc, 0.0)


def _tc_final(selfpart, agg2, W_msg):
    return pl.pallas_call(
        _final_body,
        grid=(_N // _TC_ROWS,),
        in_specs=[
            pl.BlockSpec((_TC_ROWS, _D), lambda i: (i, 0)),
            pl.BlockSpec((_NC, _TC_ROWS, _D), lambda i: (0, i, 0)),
            pl.BlockSpec((_D, _D), lambda i: (0, 0)),
        ],
        out_specs=pl.BlockSpec((_TC_ROWS, _D), lambda i: (i, 0)),
        out_shape=jax.ShapeDtypeStruct((_N, _D), jnp.float32),
    )(selfpart, agg2, W_msg)


def kernel(x, edge_index, W_msg, W_self, b):
    packed = _tc_pack(edge_index.astype(jnp.int32))
    zeros_blk = jnp.zeros((_CHUNK, _D), jnp.bfloat16)
    xbf = x.astype(jnp.bfloat16)
    agg2 = _sc_segment_sum()(xbf, packed, zeros_blk).reshape(_NC, _N, _D)
    selfpart = _tc_self(x, W_self, b.reshape(1, _D))
    return _tc_final(selfpart, agg2, W_msg)
